# baseline (speedup 1.0000x reference)
import functools

import jax
import jax.numpy as jnp
from jax.experimental import pallas as pl
from jax.experimental.pallas import tpu as pltpu
from jax.experimental.pallas import tpu_sc as plsc

_D0, _D1, _D2 = 2, 4, 10

_mesh = plsc.ScalarSubcoreMesh(axis_name="c", num_cores=1)


@functools.partial(
    pl.kernel,
    mesh=_mesh,
    compiler_params=pltpu.CompilerParams(use_tc_tiling_on_sc=True),
    out_type=jax.ShapeDtypeStruct((_D0, _D1, _D2), jnp.float32),
    scratch_types=[
        pltpu.SMEM((_D0, _D1, 1), jnp.float32),
        pltpu.SMEM((_D0, _D1, _D2), jnp.float32),
        pltpu.SMEM((_D0, _D1, _D2), jnp.float32),
        pltpu.SemaphoreType.DMA,
    ],
)
def _sc_broadcast_mul(a_hbm, b_hbm, out_hbm, a_s, b_s, o_s, sem):
    cp_a = pltpu.make_async_copy(a_hbm, a_s, sem)
    cp_b = pltpu.make_async_copy(b_hbm, b_s, sem)
    cp_a.start()
    cp_b.start()
    cp_a.wait()
    cp_b.wait()
    for i in range(_D0):
        for j in range(_D1):
            aij = a_s[i, j, 0]
            for k in range(_D2):
                o_s[i, j, k] = aij * b_s[i, j, k]
    pltpu.sync_copy(o_s, out_hbm)


def kernel(x, a_dense, b_dense):
    del x
    return _sc_broadcast_mul(a_dense, b_dense)


# speedup vs baseline: 1.0146x; 1.0146x over previous
import functools

import jax
import jax.numpy as jnp
from jax.experimental import pallas as pl
from jax.experimental.pallas import tpu as pltpu
from jax.experimental.pallas import tpu_sc as plsc

_D0, _D1, _D2 = 2, 4, 10

_mesh = plsc.ScalarSubcoreMesh(axis_name="c", num_cores=1)


@functools.partial(
    pl.kernel,
    mesh=_mesh,
    compiler_params=pltpu.CompilerParams(skip_device_barrier=True),
    out_type=jax.ShapeDtypeStruct((_D0, _D1, _D2), jnp.float32),
    scratch_types=[
        pltpu.SMEM((_D0, _D1, 1), jnp.float32),
        pltpu.SMEM((_D0, _D1, _D2), jnp.float32),
        pltpu.SMEM((_D0, _D1, _D2), jnp.float32),
        pltpu.SemaphoreType.DMA,
    ],
)
def _sc_broadcast_mul(a_hbm, b_hbm, out_hbm, a_s, b_s, o_s, sem):
    cp_a = pltpu.make_async_copy(a_hbm, a_s, sem)
    cp_b = pltpu.make_async_copy(b_hbm, b_s, sem)
    cp_a.start()
    cp_b.start()
    cp_a.wait()
    cp_b.wait()
    for i in range(_D0):
        for j in range(_D1):
            aij = a_s[i, j, 0]
            for k in range(_D2):
                o_s[i, j, k] = aij * b_s[i, j, k]
    pltpu.sync_copy(o_s, out_hbm)


def kernel(x, a_dense, b_dense):
    del x
    return _sc_broadcast_mul(a_dense, b_dense)
